# baseline (speedup 1.0000x reference)
import functools

import jax
import jax.numpy as jnp
from jax.experimental import pallas as pl
from jax.experimental.pallas import tpu as pltpu


def _body(grid, C, yp_ref, yt_ref, w_ref, out_ref, acc_t, acc_p):
    i = pl.program_id(0)

    @pl.when(i == 0)
    def _init():
        acc_t[...] = jnp.zeros_like(acc_t)
        acc_p[...] = jnp.zeros_like(acc_p)

    x = yp_ref[...]
    yt = yt_ref[0]
    m = jnp.max(x, axis=1, keepdims=True)
    colid = jax.lax.broadcasted_iota(jnp.int32, x.shape, 1)
    oh = colid == yt
    eq = x == m
    oh_f = jnp.where(oh, 1.0, 0.0)
    pc_f = jnp.where(oh & eq, 1.0, 0.0)
    acc_t[...] += jnp.sum(oh_f, axis=0, keepdims=True)
    acc_p[...] += jnp.sum(pc_f, axis=0, keepdims=True)

    @pl.when(i == grid - 1)
    def _fin():
        tc = acc_t[...]
        pc = acc_p[...]
        w = w_ref[...]
        acc = jnp.where(tc > 0, pc / jnp.maximum(tc, 1.0), 0.0)
        out_ref[...] = jnp.reshape(jnp.sum(acc * w) / jnp.sum(w), (1, 1))


def kernel(y_pred, y_true, weights):
    N, C = y_pred.shape
    R = 2000
    grid = N // R
    yt3 = y_true.astype(jnp.int32).reshape(grid, R, 1)
    w2 = weights.reshape(1, C)
    out = pl.pallas_call(
        functools.partial(_body, grid, C),
        grid=(grid,),
        in_specs=[
            pl.BlockSpec((R, C), lambda i: (i, 0)),
            pl.BlockSpec((1, R, 1), lambda i: (i, 0, 0)),
            pl.BlockSpec((1, C), lambda i: (0, 0)),
        ],
        out_specs=pl.BlockSpec((1, 1), lambda i: (0, 0)),
        out_shape=jax.ShapeDtypeStruct((1, 1), jnp.float32),
        scratch_shapes=[
            pltpu.VMEM((1, C), jnp.float32),
            pltpu.VMEM((1, C), jnp.float32),
        ],
        compiler_params=pltpu.CompilerParams(
            dimension_semantics=("arbitrary",),
        ),
    )(y_pred, yt3, w2)
    return out[0, 0]


# speedup vs baseline: 1.8929x; 1.8929x over previous
import functools

import jax
import jax.numpy as jnp
from jax.experimental import pallas as pl
from jax.experimental.pallas import tpu as pltpu


def _body(grid, C, yp_ref, yt_ref, w_ref, out_ref, acc_t, acc_p):
    i = pl.program_id(0)

    @pl.when(i == 0)
    def _init():
        acc_t[...] = jnp.zeros_like(acc_t)
        acc_p[...] = jnp.zeros_like(acc_p)

    R = yp_ref.shape[0]
    xt = yp_ref[...].T
    yt = yt_ref[0]
    mt = jnp.max(xt, axis=0, keepdims=True)
    rowid = jax.lax.broadcasted_iota(jnp.int32, (C, R), 0)
    oh_f = jnp.where(rowid == yt, 1.0, 0.0)
    v_true = jnp.sum(oh_f * xt, axis=0, keepdims=True)
    correct_f = jnp.where(v_true == mt, 1.0, 0.0)
    rhs = jnp.concatenate(
        [jnp.ones((1, R), jnp.float32), correct_f], axis=0
    ).T
    counts = jax.lax.dot_general(
        oh_f, rhs, (((1,), (0,)), ((), ())),
        preferred_element_type=jnp.float32,
    )
    acc_t[...] += counts[:, 0:1]
    acc_p[...] += counts[:, 1:2]

    @pl.when(i == grid - 1)
    def _fin():
        tc = acc_t[...]
        pc = acc_p[...]
        w = w_ref[...]
        acc = jnp.where(tc > 0, pc / jnp.maximum(tc, 1.0), 0.0)
        out_ref[...] = jnp.reshape(jnp.sum(acc * w) / jnp.sum(w), (1, 1))


def kernel(y_pred, y_true, weights):
    N, C = y_pred.shape
    R = 2000
    grid = N // R
    yt3 = y_true.astype(jnp.int32).reshape(grid, 1, R)
    w2 = weights.reshape(C, 1)
    out = pl.pallas_call(
        functools.partial(_body, grid, C),
        grid=(grid,),
        in_specs=[
            pl.BlockSpec((R, C), lambda i: (i, 0)),
            pl.BlockSpec((1, 1, R), lambda i: (i, 0, 0)),
            pl.BlockSpec((C, 1), lambda i: (0, 0)),
        ],
        out_specs=pl.BlockSpec((1, 1), lambda i: (0, 0)),
        out_shape=jax.ShapeDtypeStruct((1, 1), jnp.float32),
        scratch_shapes=[
            pltpu.VMEM((C, 1), jnp.float32),
            pltpu.VMEM((C, 1), jnp.float32),
        ],
        compiler_params=pltpu.CompilerParams(
            dimension_semantics=("arbitrary",),
        ),
    )(y_pred, yt3, w2)
    return out[0, 0]
